# Initial kernel scaffold; baseline (speedup 1.0000x reference)
import jax
import jax.numpy as jnp
from jax.experimental import pallas as pl


def kernel(coords, factors, weights, bias):
    raise NotImplementedError("write your pallas kernel here")


# speedup vs baseline: 1.0172x; 1.0172x over previous
import functools

import jax
import jax.numpy as jnp
from jax import lax
from jax.experimental import pallas as pl
from jax.experimental.pallas import tpu as pltpu
from jax.experimental.pallas import tpu_sc as plsc

_L = 16
_NC = 2
_NS = 16
_NW = _NC * _NS


def _cp_forward(table, coords_flat, w_splat, bias_splat, *, B, H, V, R):
    BPW = B // _NW
    C = 64
    NCHUNK = BPW // C
    NG = C // _L

    mesh = plsc.VectorSubcoreMesh(core_axis_name="c", subcore_axis_name="s")

    @functools.partial(
        pl.kernel,
        out_type=jax.ShapeDtypeStruct((B,), jnp.float32),
        mesh=mesh,
        compiler_params=pltpu.CompilerParams(
            needs_layout_passes=False, use_tc_tiling_on_sc=False),
        scratch_types=dict(
            coords_v=pltpu.VMEM((BPW * H,), jnp.int32),
            idx0=pltpu.VMEM((H, C), jnp.int32),
            idx1=pltpu.VMEM((H, C), jnp.int32),
            buf0=pltpu.VMEM((H * C, R), jnp.float32),
            buf1=pltpu.VMEM((H * C, R), jnp.float32),
            w_v=pltpu.VMEM((R, _L), jnp.float32),
            b_v=pltpu.VMEM((_L,), jnp.float32),
            out_v=pltpu.VMEM((BPW,), jnp.float32),
            sem0=pltpu.SemaphoreType.DMA,
            sem1=pltpu.SemaphoreType.DMA,
        ),
    )
    def run(table_hbm, coords_hbm, w_hbm, b_hbm, out_hbm, *, coords_v,
            idx0, idx1, buf0, buf1, w_v, b_v, out_v, sem0, sem1):
        wid = lax.axis_index("s") * _NC + lax.axis_index("c")
        base = wid * BPW

        pltpu.sync_copy(coords_hbm.at[pl.ds(base * H, BPW * H)], coords_v)
        pltpu.sync_copy(w_hbm, w_v)
        pltpu.sync_copy(b_hbm, b_v)

        iota = lax.iota(jnp.int32, _L)
        iota_h = iota * H

        idx_refs = (idx0, idx1)
        buf_refs = (buf0, buf1)
        sems = (sem0, sem1)

        def build_and_fire(c, s):
            idx_r, buf_r, sem = idx_refs[s], buf_refs[s], sems[s]
            for m in range(H):
                for j in range(C // _L):
                    flat0 = (c * C + j * _L) * H + m
                    cv = plsc.load_gather(coords_v, [iota_h + flat0])
                    idx_r[m, pl.ds(j * _L, _L)] = cv + m * V
                pltpu.async_copy(
                    table_hbm.at[idx_r.at[m]],
                    buf_r.at[pl.ds(m * C, C)],
                    sem,
                )

        def drain(s):
            idx_r, buf_r, sem = idx_refs[s], buf_refs[s], sems[s]
            for m in range(H):
                pltpu.make_async_copy(
                    table_hbm.at[idx_r.at[m]],
                    buf_r.at[pl.ds(m * C, C)],
                    sem,
                ).wait()

        def compute(c, s):
            buf_r = buf_refs[s]

            def load_rows(m, row16, r):
                return plsc.load_gather(
                    buf_r, [m * C + row16, jnp.full((_L,), r, jnp.int32)])

            for g in range(NG):
                row16 = iota + g * _L
                acc = tuple(load_rows(0, row16, r) for r in range(R))

                def mbody(m, acc):
                    return tuple(acc[r] * load_rows(m, row16, r)
                                 for r in range(R))

                acc = lax.fori_loop(1, H, mbody, acc)
                y = b_v[...]
                for r in range(R):
                    y = y + w_v[r, :] * acc[r]
                out_v[pl.ds(c * C + g * _L, _L)] = y

        build_and_fire(0, 0)

        @pl.loop(0, NCHUNK, step=2)
        def _(cc):
            build_and_fire(cc + 1, 1)
            drain(0)
            compute(cc, 0)

            @pl.when(cc + 2 < NCHUNK)
            def _():
                build_and_fire(cc + 2, 0)

            drain(1)
            compute(cc + 1, 1)

        pltpu.sync_copy(out_v, out_hbm.at[pl.ds(base, BPW)])

    return run(table, coords_flat, w_splat, bias_splat)


def kernel(coords, factors, weights, bias):
    B, H = coords.shape
    _, V, R = factors.shape
    table = factors.reshape(H * V, R)
    coords_flat = coords.astype(jnp.int32).reshape(-1)
    w_splat = jnp.broadcast_to(
        weights.astype(jnp.float32)[:, None], (R, _L))
    bias_splat = jnp.broadcast_to(bias.astype(jnp.float32), (_L,))
    return _cp_forward(table, coords_flat, w_splat, bias_splat,
                       B=B, H=H, V=V, R=R)
